# baseline (speedup 1.0000x reference)
import functools

import jax
import jax.numpy as jnp
from jax import lax
from jax.experimental import pallas as pl
from jax.experimental.pallas import tpu as pltpu
from jax.experimental.pallas import tpu_sc as plsc

B = 16384
NREF = 8
S = NREF + 1
DIM = 32
BETA = 10.0
GAMMA = 0.001

NC = 2
NS = 16
NW = NC * NS
ROWS_PW = B // NW
CHUNK = 128
NCH = ROWS_PW // CHUNK

_mesh = plsc.VectorSubcoreMesh(core_axis_name="core", subcore_axis_name="sub")


@functools.partial(
    pl.kernel,
    mesh=_mesh,
    compiler_params=pltpu.CompilerParams(
        needs_layout_passes=False, use_tc_tiling_on_sc=False
    ),
    out_type=jax.ShapeDtypeStruct((NREF, B), jnp.float32),
    scratch_types=[
        pltpu.VMEM((S * CHUNK,), jnp.int32),
        pltpu.VMEM((S * CHUNK,), jnp.int32),
        pltpu.VMEM((S * CHUNK, DIM), jnp.float32),
        pltpu.VMEM((S * CHUNK, DIM), jnp.float32),
        pltpu.VMEM((NREF, CHUNK), jnp.float32),
        pltpu.VMEM((NREF, CHUNK), jnp.float32),
        pltpu.SemaphoreType.DMA,
        pltpu.SemaphoreType.DMA,
        pltpu.SemaphoreType.DMA,
        pltpu.SemaphoreType.DMA,
    ],
)
def _rank_kernel(
    stim_hbm, table_hbm, out_hbm,
    idx_v0, idx_v1, rows_v0, rows_v1, out_v0, out_v1,
    gsem0, gsem1, osem0, osem1,
):
    wid = lax.axis_index("sub") * NC + lax.axis_index("core")
    lanes = lax.iota(jnp.int32, 16)
    idx_bufs = (idx_v0, idx_v1)
    rows_bufs = (rows_v0, rows_v1)
    out_bufs = (out_v0, out_v1)
    gsems = (gsem0, gsem1)
    osems = (osem0, osem1)

    def stage(c):
        row0 = wid * ROWS_PW + c * CHUNK
        idx_v = idx_bufs[c % 2]
        for j in range(S):
            pltpu.sync_copy(
                stim_hbm.at[j, pl.ds(row0, CHUNK)],
                idx_v.at[pl.ds(j * CHUNK, CHUNK)],
            )
        return pltpu.async_copy(
            table_hbm.at[idx_v], rows_bufs[c % 2], gsems[c % 2]
        )

    def compute(c):
        idx_v = idx_bufs[c % 2]
        rows_v = rows_bufs[c % 2]
        out_v = out_bufs[c % 2]

        def group_body(t, inner_carry):
            b = t * 16 + lanes
            acc = [jnp.zeros((16,), jnp.float32) for _ in range(NREF)]
            for k in range(DIM):
                dcol = (lanes + k) & (DIM - 1)
                zq = plsc.load_gather(rows_v, [b, dcol])
                for r in range(NREF):
                    zr = plsc.load_gather(rows_v, [b + (r + 1) * CHUNK, dcol])
                    df = zq - zr
                    acc[r] = acc[r] + df * df
            total = jnp.full((16,), 1e-16, jnp.float32)
            sv = []
            for r in range(NREF):
                x = acc[r]
                i = jnp.full((16,), 0x5F3759DF, jnp.int32) - (
                    plsc.bitcast(x, jnp.int32) >> 1
                )
                y = plsc.bitcast(i, jnp.float32)
                y = y * (1.5 - 0.5 * x * y * y)
                y = y * (1.5 - 0.5 * x * y * y)
                y = y * (1.5 - 0.5 * x * y * y)
                dist = x * y
                sval = jnp.exp(-BETA * dist) + GAMMA
                stim_r = idx_v[pl.ds((r + 1) * CHUNK + t * 16, 16)]
                sval = jnp.where(stim_r != 0, sval, 0.0)
                sv.append(sval)
                total = total + sval
            inv = 1.0 / total
            for r in range(NREF):
                out_v[r, pl.ds(t * 16, 16)] = sv[r] * inv
            return inner_carry

        lax.fori_loop(0, CHUNK // 16, group_body, 0)

    def flush(c):
        row0 = wid * ROWS_PW + c * CHUNK
        out_v = out_bufs[c % 2]
        copies = []
        for r in range(NREF):
            copies.append(
                pltpu.async_copy(
                    out_v.at[r], out_hbm.at[r, pl.ds(row0, CHUNK)], osems[c % 2]
                )
            )
        return copies

    pending_out = []
    gather = stage(0)
    for c in range(NCH):
        nxt = stage(c + 1) if c + 1 < NCH else None
        gather.wait()
        compute(c)
        for cp in pending_out:
            cp.wait()
        pending_out = flush(c)
        gather = nxt
    for cp in pending_out:
        cp.wait()


def kernel(stimulus_set, percept_table):
    out = _rank_kernel(stimulus_set.T, percept_table)
    return out.T


# speedup vs baseline: 1.0150x; 1.0150x over previous
import functools

import jax
import jax.numpy as jnp
from jax import lax
from jax.experimental import pallas as pl
from jax.experimental.pallas import tpu as pltpu
from jax.experimental.pallas import tpu_sc as plsc

B = 16384
NREF = 8
S = NREF + 1
DIM = 32
BETA = 10.0
GAMMA = 0.001

NC = 2
NS = 16
NW = NC * NS
ROWS_PW = B // NW
CHUNK = 128
NCH = ROWS_PW // CHUNK

_mesh = plsc.VectorSubcoreMesh(core_axis_name="core", subcore_axis_name="sub")


@functools.partial(
    pl.kernel,
    mesh=_mesh,
    compiler_params=pltpu.CompilerParams(
        needs_layout_passes=False, use_tc_tiling_on_sc=False
    ),
    out_type=jax.ShapeDtypeStruct((NREF, B), jnp.float32),
    scratch_types=[
        pltpu.VMEM((S * CHUNK,), jnp.int32),
        pltpu.VMEM((S * CHUNK,), jnp.int32),
        pltpu.VMEM((S * CHUNK, DIM), jnp.float32),
        pltpu.VMEM((S * CHUNK, DIM), jnp.float32),
        pltpu.VMEM((NREF, CHUNK), jnp.float32),
        pltpu.VMEM((NREF, CHUNK), jnp.float32),
        pltpu.SemaphoreType.DMA,
        pltpu.SemaphoreType.DMA,
        pltpu.SemaphoreType.DMA,
        pltpu.SemaphoreType.DMA,
    ],
)
def _rank_kernel(
    stim_hbm, table_hbm, out_hbm,
    idx_v0, idx_v1, rows_v0, rows_v1, out_v0, out_v1,
    gsem0, gsem1, osem0, osem1,
):
    wid = lax.axis_index("sub") * NC + lax.axis_index("core")
    lanes = lax.iota(jnp.int32, 16)
    idx_bufs = (idx_v0, idx_v1)
    rows_bufs = (rows_v0, rows_v1)
    out_bufs = (out_v0, out_v1)
    gsems = (gsem0, gsem1)
    osems = (osem0, osem1)

    def stage(c):
        row0 = wid * ROWS_PW + c * CHUNK
        idx_v = idx_bufs[c % 2]
        for j in range(S):
            pltpu.sync_copy(
                stim_hbm.at[j, pl.ds(row0, CHUNK)],
                idx_v.at[pl.ds(j * CHUNK, CHUNK)],
            )
        for g in range(S * CHUNK // 16):
            sl = pl.ds(g * 16, 16)
            idx_v[sl] = idx_v[sl] * 4
        return pltpu.async_copy(
            table_hbm.at[idx_v], rows_bufs[c % 2], gsems[c % 2]
        )

    def compute(c):
        idx_v = idx_bufs[c % 2]
        rows_v = rows_bufs[c % 2]
        out_v = out_bufs[c % 2]

        def group_body(t, inner_carry):
            b = t * 16 + lanes
            acc = [jnp.zeros((16,), jnp.float32) for _ in range(NREF)]
            for k in range(DIM):
                dcol = (lanes + k) & (DIM - 1)
                zq = plsc.load_gather(rows_v, [b, dcol])
                for r in range(NREF):
                    zr = plsc.load_gather(rows_v, [b + (r + 1) * CHUNK, dcol])
                    df = zq - zr
                    acc[r] = acc[r] + df * df
            total = jnp.full((16,), 1e-16, jnp.float32)
            sv = []
            for r in range(NREF):
                x = acc[r]
                i = jnp.full((16,), 0x5F3759DF, jnp.int32) - (
                    plsc.bitcast(x, jnp.int32) >> 1
                )
                y = plsc.bitcast(i, jnp.float32)
                y = y * (1.5 - 0.5 * x * y * y)
                y = y * (1.5 - 0.5 * x * y * y)
                y = y * (1.5 - 0.5 * x * y * y)
                dist = x * y
                sval = jnp.exp(-BETA * dist) + GAMMA
                stim_r = idx_v[pl.ds((r + 1) * CHUNK + t * 16, 16)]
                sval = jnp.where(stim_r != 0, sval, 0.0)
                sv.append(sval)
                total = total + sval
            inv = 1.0 / total
            for r in range(NREF):
                out_v[r, pl.ds(t * 16, 16)] = sv[r] * inv
            return inner_carry

        lax.fori_loop(0, CHUNK // 16, group_body, 0)

    def flush(c):
        row0 = wid * ROWS_PW + c * CHUNK
        out_v = out_bufs[c % 2]
        copies = []
        for r in range(NREF):
            copies.append(
                pltpu.async_copy(
                    out_v.at[r], out_hbm.at[r, pl.ds(row0, CHUNK)], osems[c % 2]
                )
            )
        return copies

    pending_out = []
    gather = stage(0)
    for c in range(NCH):
        nxt = stage(c + 1) if c + 1 < NCH else None
        gather.wait()
        compute(c)
        for cp in pending_out:
            cp.wait()
        pending_out = flush(c)
        gather = nxt
    for cp in pending_out:
        cp.wait()


def kernel(stimulus_set, percept_table):
    tab = jnp.pad(percept_table, ((0, 0), (0, 128 - DIM)))
    tab = tab.reshape((B * 0 + 400004, DIM))
    out = _rank_kernel(stimulus_set.T, tab)
    return out.T
